# baseline (speedup 1.0000x reference)
import functools

import jax
import jax.numpy as jnp
from jax import lax
from jax.experimental import pallas as pl
from jax.experimental.pallas import tpu as pltpu
from jax.experimental.pallas import tpu_sc as plsc

NC = 2
NS = 16
NW = NC * NS
G = 640
NB = 4


@functools.cache
def _build(vocab, dim, n_total):
    per_w = n_total // NW
    t_steps = per_w // G
    assert per_w * NW == n_total and t_steps * G == per_w
    assert t_steps % NB == 0 and t_steps >= 2 * NB

    mesh = plsc.VectorSubcoreMesh(core_axis_name="c", subcore_axis_name="s")

    @functools.partial(
        pl.kernel,
        out_type=jax.ShapeDtypeStruct((NW, t_steps, G, dim), jnp.float32),
        mesh=mesh,
        compiler_params=pltpu.CompilerParams(use_tc_tiling_on_sc=False),
        scratch_types=(
            [pltpu.VMEM((t_steps, G), jnp.int32),
             pltpu.VMEM((NB, G, dim), jnp.float32)]
            + [pltpu.SemaphoreType.DMA for _ in range(2 * NB)]
        ),
    )
    def body(idx_hbm, table_hbm, out_hbm, idx_v, rows_v, *sems):
        wid = lax.axis_index("s") * NC + lax.axis_index("c")
        gsems = sems[:NB]
        osems = sems[NB:]

        pltpu.sync_copy(idx_hbm.at[wid], idx_v)

        def start_gather(t, ph):
            pltpu.async_copy(table_hbm.at[idx_v.at[t]], rows_v.at[ph],
                             gsems[ph])

        def wait_gather(ph):
            pltpu.make_async_copy(table_hbm.at[idx_v.at[0]], rows_v.at[ph],
                                  gsems[ph]).wait()

        def start_out(t, ph):
            pltpu.async_copy(rows_v.at[ph], out_hbm.at[wid, t], osems[ph])

        def wait_out(ph):
            pltpu.make_async_copy(rows_v.at[ph], out_hbm.at[wid, 0],
                                  osems[ph]).wait()

        for p in range(NB):
            start_gather(p, p)
        wait_gather(0)
        start_out(0, 0)

        @pl.loop(1, t_steps // NB)
        def _(u):
            for ph in range(NB):
                t = NB * u + ph
                wait_out(ph)
                start_gather(t, ph)
                dph = (ph + 1) % NB
                wait_gather(dph)
                start_out(t - (NB - 1), dph)

        for e in range(t_steps - NB + 1, t_steps):
            wait_gather(e % NB)
            start_out(e, e % NB)
        for ph in range(NB):
            wait_out(ph)

    return body


def kernel(inputs, weight):
    b, l = inputs.shape
    vocab, dim = weight.shape
    n_total = b * l
    t_steps = n_total // NW // G

    idx3 = inputs.reshape(NW, t_steps, G)
    out = _build(vocab, dim, n_total)(idx3, weight)
    return out.reshape(b, l, dim)
